# Initial kernel scaffold; baseline (speedup 1.0000x reference)
import jax
import jax.numpy as jnp
from jax.experimental import pallas as pl


def kernel(user_emb, item_emb, edge_index, edge_weight, users, items):
    raise NotImplementedError("write your pallas kernel here")


# speedup vs baseline: 5.3985x; 5.3985x over previous
import jax
import jax.numpy as jnp
from jax import lax
from jax.experimental import pallas as pl
from jax.experimental.pallas import tpu as pltpu
from jax.experimental.pallas import tpu_sc as plsc

_N_USERS = 30000
_N_NODES = 50000
_E = 800000
_H = 32
_NSUB = 16
_EPAD = 819200
_ET = _EPAD // _NSUB
_CH = 512
_NCH = _ET // _CH
_SUB = 128
_NJ = _CH // _SUB
_RH = 51200
_TR = _RH // _NSUB
_ZR = 160
_WR = 400
_B = 4096


def _sc_body(x0, srcp, dstp, wp, usr, itm, x1, x2, x3, gpart,
             acc, rows, sidx, didx, wbuf, zbuf, gbuf, gsem, ssem):
  c = lax.axis_index("c")
  s = lax.axis_index("s")
  coff = c * _RH
  zv = jnp.zeros((16,), jnp.float32)

  def zb(r, carry):
    zbuf[r, pl.ds(0, 16)] = zv
    zbuf[r, pl.ds(16, 16)] = zv
    return carry

  lax.fori_loop(0, _ZR, zb, 0)
  rs = s * _TR
  for j in range(_TR // _ZR):
    pltpu.sync_copy(zbuf, acc.at[pl.ds(rs + j * _ZR, _ZR)])
  plsc.subcore_barrier()

  for xin, xout in ((x0, x1), (x1, x2), (x2, x3)):
    irow0 = s * (_ET // _SUB)

    def chunk(ci, carry, xin=xin):
      ir = irow0 + ci * _NJ
      pltpu.sync_copy(srcp.at[pl.ds(ir, _NJ)], sidx)
      pltpu.sync_copy(dstp.at[pl.ds(ir, _NJ)], didx)
      eb = s * _ET + ci * _CH
      pltpu.sync_copy(wp.at[pl.ds(eb, _CH)], wbuf)

      def offr(r, cy):
        for k in range(8):
          sidx[r, pl.ds(k * 16, 16)] = sidx[r, pl.ds(k * 16, 16)] + coff
        return cy

      lax.fori_loop(0, _NJ, offr, 0)
      cps = [pltpu.async_copy(xin.at[sidx.at[j]],
                              rows.at[pl.ds(j * _SUB, _SUB)], gsem)
             for j in range(_NJ)]
      for cp in cps:
        cp.wait()

      def scale(g, cy):
        wv = wbuf[pl.ds(g * 16, 16)]
        for j in range(16):
          e = g * 16 + j
          w = wv[j]
          rows[e, pl.ds(0, 16)] = rows[e, pl.ds(0, 16)] * w
          rows[e, pl.ds(16, 16)] = rows[e, pl.ds(16, 16)] * w
        return cy

      lax.fori_loop(0, _CH // 16, scale, 0, unroll=False)
      cps2 = [pltpu.async_copy(rows.at[pl.ds(j * _SUB, _SUB)],
                               acc.at[didx.at[j]], ssem, add=True)
              for j in range(_NJ)]
      for cp in cps2:
        cp.wait()
      return carry

    lax.fori_loop(0, _NCH, chunk, 0)
    plsc.subcore_barrier()
    stage = rows.at[pl.ds(0, _WR)]
    for j in range(_TR // _WR):
      pltpu.sync_copy(acc.at[pl.ds(rs + j * _WR, _WR)], stage)
      pltpu.sync_copy(stage, xout.at[pl.ds(coff + rs + j * _WR, _WR)])
    for j in range(_TR // _ZR):
      pltpu.sync_copy(zbuf, acc.at[pl.ds(rs + j * _ZR, _ZR)])
    plsc.subcore_barrier()

  pltpu.sync_copy(usr.at[pl.ds(s * 2, 2)], sidx.at[pl.ds(0, 2)])
  pltpu.sync_copy(itm.at[pl.ds(s * 2, 2)], didx.at[pl.ds(0, 2)])
  for q in range(2):
    for k in range(8):
      sidx[q, pl.ds(k * 16, 16)] = sidx[q, pl.ds(k * 16, 16)] + coff
      didx[q, pl.ds(k * 16, 16)] = didx[q, pl.ds(k * 16, 16)] + (coff + _N_USERS)
  xs = (x0, x1, x2, x3)
  lane = lax.iota(jnp.int32, 16)
  perms = [(lane + sh) & 15 for sh in (8, 4, 2, 1)]
  gdims = lax.GatherDimensionNumbers(
      offset_dims=(), collapsed_slice_dims=(0,), start_index_map=(0,))

  def _hsum(v):
    for p in perms:
      v = v + lax.gather(v, p[:, None], gdims, (1,),
                         mode=lax.GatherScatterMode.PROMISE_IN_BOUNDS)
    return v

  for q in range(2):
    cps = [pltpu.async_copy(xs[l].at[sidx.at[q]],
                            rows.at[pl.ds((l + 1) * _SUB, _SUB)], gsem)
           for l in range(4)]
    for cp in cps:
      cp.wait()

    def usum(g, cy):
      for j in range(8):
        p = g * 8 + j
        for h in (0, 16):
          v = (rows[_SUB + p, pl.ds(h, 16)] + rows[2 * _SUB + p, pl.ds(h, 16)]
               + rows[3 * _SUB + p, pl.ds(h, 16)]
               + rows[4 * _SUB + p, pl.ds(h, 16)])
          rows[p, pl.ds(h, 16)] = v
      return cy

    lax.fori_loop(0, 16, usum, 0)
    cps = [pltpu.async_copy(xs[l].at[didx.at[q]],
                            rows.at[pl.ds((l + 1) * _SUB, _SUB)], gsem)
           for l in range(4)]
    for cp in cps:
      cp.wait()

    def pair_one(t, cy, q=q):
      ilo = (rows[_SUB + t, pl.ds(0, 16)] + rows[2 * _SUB + t, pl.ds(0, 16)]
             + rows[3 * _SUB + t, pl.ds(0, 16)] + rows[4 * _SUB + t, pl.ds(0, 16)])
      ihi = (rows[_SUB + t, pl.ds(16, 16)] + rows[2 * _SUB + t, pl.ds(16, 16)]
             + rows[3 * _SUB + t, pl.ds(16, 16)] + rows[4 * _SUB + t, pl.ds(16, 16)])
      hs = _hsum(rows[t, pl.ds(0, 16)] * ilo + rows[t, pl.ds(16, 16)] * ihi)
      base = q * 128 + (t & ~15)
      av = gbuf[pl.ds(base, 16)]
      gbuf[pl.ds(base, 16)] = jnp.where(lane == (t & 15), hs * 0.0625, av)
      return cy

    lax.fori_loop(0, 128, pair_one, 0)
  pltpu.sync_copy(gbuf, gpart.at[c, 0, pl.ds(s * 256, 256)])


def _make_kernel():
  mesh = plsc.VectorSubcoreMesh(core_axis_name="c", subcore_axis_name="s")
  out_type = [
      jax.ShapeDtypeStruct((2 * _RH, _H), jnp.float32),
      jax.ShapeDtypeStruct((2 * _RH, _H), jnp.float32),
      jax.ShapeDtypeStruct((2 * _RH, _H), jnp.float32),
      jax.ShapeDtypeStruct((2, 1, _B), jnp.float32),
  ]
  scratch = [
      pltpu.VMEM_SHARED((_RH, _H), jnp.float32),
      pltpu.VMEM((5 * _SUB, _H), jnp.float32),
      pltpu.VMEM((_NJ, _SUB), jnp.int32),
      pltpu.VMEM((_NJ, _SUB), jnp.int32),
      pltpu.VMEM((_CH,), jnp.float32),
      pltpu.VMEM((_ZR, _H), jnp.float32),
      pltpu.VMEM((256,), jnp.float32),
      pltpu.SemaphoreType.DMA,
      pltpu.SemaphoreType.DMA,
  ]
  return pl.kernel(_sc_body, out_type=out_type, mesh=mesh,
                   scratch_types=scratch,
                   compiler_params=pltpu.CompilerParams(
                       use_tc_tiling_on_sc=False))


_KERNEL = _make_kernel()


@jax.jit
def kernel(user_emb, item_emb, edge_index, edge_weight, users, items):
  all_emb = jnp.concatenate([user_emb, item_emb], axis=0)
  zpad = jnp.zeros((_RH - _N_NODES, _H), jnp.float32)
  x0 = jnp.concatenate(
      [all_emb[:, :_H], zpad, all_emb[:, _H:], zpad], axis=0)
  pad = _EPAD - _E
  srcp = jnp.concatenate(
      [edge_index[0], jnp.zeros((pad,), jnp.int32)]).reshape(_EPAD // _SUB, _SUB)
  dstp = jnp.concatenate(
      [edge_index[1], jnp.zeros((pad,), jnp.int32)]).reshape(_EPAD // _SUB, _SUB)
  wp = jnp.concatenate([edge_weight, jnp.zeros((pad,), jnp.float32)])
  usr = users.reshape(_B // _SUB, _SUB)
  itm = items.reshape(_B // _SUB, _SUB)
  _, _, _, gpart = _KERNEL(x0, srcp, dstp, wp, usr, itm)
  return gpart[0, 0] + gpart[1, 0]


# speedup vs baseline: 6.7058x; 1.2422x over previous
import jax
import jax.numpy as jnp
from jax import lax
from jax.experimental import pallas as pl
from jax.experimental.pallas import tpu as pltpu
from jax.experimental.pallas import tpu_sc as plsc

_N_USERS = 30000
_N_NODES = 50000
_E = 800000
_H = 32
_NSUB = 16
_EPAD = 819200
_ET = _EPAD // _NSUB
_CH = 256
_SUB = 128
_GRP = 4
_NGRP = _ET // (_CH * _GRP)
_RH = 51200
_TR = _RH // _NSUB
_ZR = 160
_WR = 200
_B = 4096


def _sc_body(x0, srcp, dstp, wp, usr, itm, x1, x2, x3, gpart,
             acc, rows0, rows1, sidx, didx, wbuf, zbuf, gbuf, gsem, ssem):
  c = lax.axis_index("c")
  s = lax.axis_index("s")
  coff = c * _RH
  zv = jnp.zeros((16,), jnp.float32)
  rbufs = (rows0, rows1)

  def zb(r, carry):
    zbuf[r, pl.ds(0, 16)] = zv
    zbuf[r, pl.ds(16, 16)] = zv
    return carry

  lax.fori_loop(0, _ZR, zb, 0)
  rs = s * _TR
  for j in range(_TR // _ZR):
    pltpu.sync_copy(zbuf, acc.at[pl.ds(rs + j * _ZR, _ZR)])
  plsc.subcore_barrier()

  irow0 = s * (_ET // _SUB)

  def load_group(m):
    ir = irow0 + m * 8
    pltpu.sync_copy(srcp.at[pl.ds(ir, 8)], sidx)
    pltpu.sync_copy(dstp.at[pl.ds(ir, 8)], didx)
    pltpu.sync_copy(wp.at[pl.ds(s * _ET + m * _GRP * _CH, _GRP * _CH)], wbuf)

    def offr(r, cy):
      for k in range(8):
        sidx[r, pl.ds(k * 16, 16)] = sidx[r, pl.ds(k * 16, 16)] + coff
      return cy

    lax.fori_loop(0, 8, offr, 0)

  def fire_gather(xin, j, rbuf):
    for jj in range(2):
      pltpu.async_copy(xin.at[sidx.at[2 * j + jj]],
                       rbuf.at[pl.ds(jj * _SUB, _SUB)], gsem)

  def wait_gather(xin, rbuf):
    for jj in range(2):
      pltpu.make_async_copy(xin.at[pl.ds(0, _SUB)],
                            rbuf.at[pl.ds(jj * _SUB, _SUB)], gsem).wait()

  def scale_chunk(rbuf, j):
    def scale(g, cy):
      wv = wbuf[pl.ds(j * _CH + g * 16, 16)]
      for k in range(16):
        e = g * 16 + k
        w = wv[k]
        rbuf[e, pl.ds(0, 16)] = rbuf[e, pl.ds(0, 16)] * w
        rbuf[e, pl.ds(16, 16)] = rbuf[e, pl.ds(16, 16)] * w
      return cy

    lax.fori_loop(0, _CH // 16, scale, 0)

  def scatter_chunk(rbuf, j):
    cps = [pltpu.async_copy(rbuf.at[pl.ds(jj * _SUB, _SUB)],
                            acc.at[didx.at[2 * j + jj]], ssem, add=True)
           for jj in range(2)]
    for cp in cps:
      cp.wait()

  for xin, xout in ((x0, x1), (x1, x2), (x2, x3)):

    def grp_body(m, carry, xin=xin):
      for j in range(_GRP):
        rb = rbufs[j & 1]
        nrb = rbufs[(j + 1) & 1]
        if j < _GRP - 1:
          fire_gather(xin, j + 1, nrb)
        wait_gather(xin, rb)
        scale_chunk(rb, j)
        scatter_chunk(rb, j)

      @pl.when(m < _NGRP - 1)
      def _():
        load_group(m + 1)
        fire_gather(xin, 0, rows0)
      return carry

    load_group(0)
    fire_gather(xin, 0, rows0)
    lax.fori_loop(0, _NGRP, grp_body, 0)
    plsc.subcore_barrier()
    stage = rows0.at[pl.ds(0, _WR)]
    for j in range(_TR // _WR):
      pltpu.sync_copy(acc.at[pl.ds(rs + j * _WR, _WR)], stage)
      pltpu.sync_copy(stage, xout.at[pl.ds(coff + rs + j * _WR, _WR)])
    for j in range(_TR // _ZR):
      pltpu.sync_copy(zbuf, acc.at[pl.ds(rs + j * _ZR, _ZR)])
    plsc.subcore_barrier()

  pltpu.sync_copy(usr.at[pl.ds(s * 2, 2)], sidx.at[pl.ds(0, 2)])
  pltpu.sync_copy(itm.at[pl.ds(s * 2, 2)], didx.at[pl.ds(0, 2)])
  for q in range(2):
    for k in range(8):
      sidx[q, pl.ds(k * 16, 16)] = sidx[q, pl.ds(k * 16, 16)] + coff
      didx[q, pl.ds(k * 16, 16)] = didx[q, pl.ds(k * 16, 16)] + (coff + _N_USERS)
  lane = lax.iota(jnp.int32, 16)
  perms = [(lane + sh) & 15 for sh in (8, 4, 2, 1)]
  gdims = lax.GatherDimensionNumbers(
      offset_dims=(), collapsed_slice_dims=(0,), start_index_map=(0,))

  def _hsum(v):
    for p in perms:
      v = v + lax.gather(v, p[:, None], gdims, (1,),
                         mode=lax.GatherScatterMode.PROMISE_IN_BOUNDS)
    return v

  def gather_pair(idx_ref, q, xa, xb):
    cps = [pltpu.async_copy(xa.at[idx_ref.at[q]], rows1.at[pl.ds(0, _SUB)], gsem),
           pltpu.async_copy(xb.at[idx_ref.at[q]], rows1.at[pl.ds(_SUB, _SUB)], gsem)]
    for cp in cps:
      cp.wait()

  for q in range(2):
    for (idx_ref, dst0) in ((sidx, 0), (didx, _SUB)):
      gather_pair(idx_ref, q, x0, x1)

      def sum1(g, cy, dst0=dst0):
        for k in range(8):
          p = g * 8 + k
          for h in (0, 16):
            rows0[dst0 + p, pl.ds(h, 16)] = (
                rows1[p, pl.ds(h, 16)] + rows1[_SUB + p, pl.ds(h, 16)])
        return cy

      lax.fori_loop(0, 16, sum1, 0)
      gather_pair(idx_ref, q, x2, x3)

      def sum2(g, cy, dst0=dst0):
        for k in range(8):
          p = g * 8 + k
          for h in (0, 16):
            rows0[dst0 + p, pl.ds(h, 16)] = (
                rows0[dst0 + p, pl.ds(h, 16)]
                + rows1[p, pl.ds(h, 16)] + rows1[_SUB + p, pl.ds(h, 16)])
        return cy

      lax.fori_loop(0, 16, sum2, 0)

    def pair_one(t, cy, q=q):
      hs = _hsum(rows0[t, pl.ds(0, 16)] * rows0[_SUB + t, pl.ds(0, 16)]
                 + rows0[t, pl.ds(16, 16)] * rows0[_SUB + t, pl.ds(16, 16)])
      base = q * 128 + (t & ~15)
      av = gbuf[pl.ds(base, 16)]
      gbuf[pl.ds(base, 16)] = jnp.where(lane == (t & 15), hs * 0.0625, av)
      return cy

    lax.fori_loop(0, 128, pair_one, 0)
  pltpu.sync_copy(gbuf, gpart.at[c, 0, pl.ds(s * 256, 256)])


def _make_kernel():
  mesh = plsc.VectorSubcoreMesh(core_axis_name="c", subcore_axis_name="s")
  out_type = [
      jax.ShapeDtypeStruct((2 * _RH, _H), jnp.float32),
      jax.ShapeDtypeStruct((2 * _RH, _H), jnp.float32),
      jax.ShapeDtypeStruct((2 * _RH, _H), jnp.float32),
      jax.ShapeDtypeStruct((2, 1, _B), jnp.float32),
  ]
  scratch = [
      pltpu.VMEM_SHARED((_RH, _H), jnp.float32),
      pltpu.VMEM((_CH, _H), jnp.float32),
      pltpu.VMEM((_CH, _H), jnp.float32),
      pltpu.VMEM((8, _SUB), jnp.int32),
      pltpu.VMEM((8, _SUB), jnp.int32),
      pltpu.VMEM((_GRP * _CH,), jnp.float32),
      pltpu.VMEM((_ZR, _H), jnp.float32),
      pltpu.VMEM((256,), jnp.float32),
      pltpu.SemaphoreType.DMA,
      pltpu.SemaphoreType.DMA,
  ]
  return pl.kernel(_sc_body, out_type=out_type, mesh=mesh,
                   scratch_types=scratch,
                   compiler_params=pltpu.CompilerParams(
                       use_tc_tiling_on_sc=False))


_KERNEL = _make_kernel()


@jax.jit
def kernel(user_emb, item_emb, edge_index, edge_weight, users, items):
  all_emb = jnp.concatenate([user_emb, item_emb], axis=0)
  zpad = jnp.zeros((_RH - _N_NODES, _H), jnp.float32)
  x0 = jnp.concatenate(
      [all_emb[:, :_H], zpad, all_emb[:, _H:], zpad], axis=0)
  pad = _EPAD - _E
  srcp = jnp.concatenate(
      [edge_index[0], jnp.zeros((pad,), jnp.int32)]).reshape(_EPAD // _SUB, _SUB)
  dstp = jnp.concatenate(
      [edge_index[1], jnp.zeros((pad,), jnp.int32)]).reshape(_EPAD // _SUB, _SUB)
  wp = jnp.concatenate([edge_weight, jnp.zeros((pad,), jnp.float32)])
  usr = users.reshape(_B // _SUB, _SUB)
  itm = items.reshape(_B // _SUB, _SUB)
  _, _, _, gpart = _KERNEL(x0, srcp, dstp, wp, usr, itm)
  return gpart[0, 0] + gpart[1, 0]


# speedup vs baseline: 7.5835x; 1.1309x over previous
import jax
import jax.numpy as jnp
from jax import lax
from jax.experimental import pallas as pl
from jax.experimental.pallas import tpu as pltpu
from jax.experimental.pallas import tpu_sc as plsc

_N_USERS = 30000
_N_NODES = 50000
_E = 800000
_H = 32
_NSUB = 16
_EPAD = 819200
_ET = _EPAD // _NSUB
_CH = 128
_GCH = 8
_NB = 25
_RH = 51200
_TR = _RH // _NSUB
_B = 4096


def _sc_body(x0, srcp, dstp, wp, usr, itm, zer, x1, x2, x3, gpart,
             acc, r0, r1, r2, r3,
             sidxa, didxa, wbufa, sidxb, didxb, wbufb, gbuf,
             gsem, ssem, isem):
  c = lax.axis_index("c")
  s = lax.axis_index("s")
  coff = c * _RH
  rbufs = (r0, r1, r2, r3)
  rs = s * _TR
  irow0 = s * (_ET // _CH)

  pltpu.sync_copy(zer, acc.at[pl.ds(rs, _TR)])
  plsc.subcore_barrier()

  def fire_group_load(m, sb, db, wb):
    ir = irow0 + m * _GCH
    pltpu.async_copy(srcp.at[pl.ds(ir, _GCH)], sb, isem)
    pltpu.async_copy(dstp.at[pl.ds(ir, _GCH)], db, isem)
    pltpu.async_copy(wp.at[pl.ds(s * _ET + m * _GCH * _CH, _GCH * _CH)], wb, isem)

  def wait_group_load(sb, db, wb):
    pltpu.make_async_copy(srcp.at[pl.ds(0, _GCH)], sb, isem).wait()
    pltpu.make_async_copy(dstp.at[pl.ds(0, _GCH)], db, isem).wait()
    pltpu.make_async_copy(wp.at[pl.ds(0, _GCH * _CH)], wb, isem).wait()

  def offset_group(sb):
    def offr(r, cy):
      for k in range(8):
        sb[r, pl.ds(k * 16, 16)] = sb[r, pl.ds(k * 16, 16)] + coff
      return cy

    lax.fori_loop(0, _GCH, offr, 0)

  def load_group_sync(m, sb, db, wb):
    fire_group_load(m, sb, db, wb)
    wait_group_load(sb, db, wb)
    offset_group(sb)

  def fire_g(xin, sb, r, rbuf):
    pltpu.async_copy(xin.at[sb.at[r]], rbuf, gsem)

  def wait_g(xin, rbuf):
    pltpu.make_async_copy(xin.at[pl.ds(0, _CH)], rbuf, gsem).wait()

  def scale_chunk(rbuf, wb, woff):
    def scale(g, cy):
      wv = wb[pl.ds(woff + g * 16, 16)]
      for k in range(16):
        e = g * 16 + k
        w = wv[k]
        rbuf[e, pl.ds(0, 16)] = rbuf[e, pl.ds(0, 16)] * w
        rbuf[e, pl.ds(16, 16)] = rbuf[e, pl.ds(16, 16)] * w
      return cy

    lax.fori_loop(0, _CH // 16, scale, 0)

  def fire_scatter(rbuf, db, r):
    pltpu.async_copy(rbuf, acc.at[db.at[r]], ssem, add=True)

  def wait_scatter():
    pltpu.make_async_copy(r0, acc.at[pl.ds(0, _CH)], ssem).wait()

  for xin, xout in ((x0, x1), (x1, x2), (x2, x3)):

    def body(b, carry, xin=xin):
      for cc in range(16):
        rb = rbufs[cc % 4]
        grp_b = cc >= 8
        sb, db, wb = (sidxb, didxb, wbufb) if grp_b else (sidxa, didxa, wbufa)
        row = cc % 8
        if cc == 2:
          fire_group_load(2 * b + 1, sidxb, didxb, wbufb)
        wait_g(xin, rb)
        scale_chunk(rb, wb, row * _CH)
        if cc == 0:
          @pl.when(b > 0)
          def _():
            wait_scatter()
        else:
          wait_scatter()
        fire_scatter(rb, db, row)
        if cc == 5:
          wait_group_load(sidxb, didxb, wbufb)
          offset_group(sidxb)
        if cc == 10:
          @pl.when(b < _NB - 1)
          def _():
            fire_group_load(2 * b + 2, sidxa, didxa, wbufa)
        if cc == 13:
          @pl.when(b < _NB - 1)
          def _():
            wait_group_load(sidxa, didxa, wbufa)
            offset_group(sidxa)
        nrb = rbufs[(cc + 2) % 4]
        if cc < 6:
          fire_g(xin, sidxa, cc + 2, nrb)
        elif cc < 14:
          fire_g(xin, sidxb, cc - 6, nrb)
        else:
          @pl.when(b < _NB - 1)
          def _(cc=cc, nrb=nrb):
            fire_g(xin, sidxa, cc - 14, nrb)
      return carry

    load_group_sync(0, sidxa, didxa, wbufa)
    fire_g(xin, sidxa, 0, r0)
    fire_g(xin, sidxa, 1, r1)
    lax.fori_loop(0, _NB, body, 0)
    wait_scatter()
    plsc.subcore_barrier()
    pltpu.sync_copy(acc.at[pl.ds(rs, _TR)], xout.at[pl.ds(coff + rs, _TR)])
    pltpu.sync_copy(zer, acc.at[pl.ds(rs, _TR)])
    plsc.subcore_barrier()

  pltpu.sync_copy(usr.at[pl.ds(s * 2, 2)], sidxa.at[pl.ds(0, 2)])
  pltpu.sync_copy(itm.at[pl.ds(s * 2, 2)], didxa.at[pl.ds(0, 2)])
  for q in range(2):
    for k in range(8):
      sidxa[q, pl.ds(k * 16, 16)] = sidxa[q, pl.ds(k * 16, 16)] + coff
      didxa[q, pl.ds(k * 16, 16)] = didxa[q, pl.ds(k * 16, 16)] + (coff + _N_USERS)
  lane = lax.iota(jnp.int32, 16)
  perms = [(lane + sh) & 15 for sh in (8, 4, 2, 1)]
  gdims = lax.GatherDimensionNumbers(
      offset_dims=(), collapsed_slice_dims=(0,), start_index_map=(0,))

  def _hsum(v):
    for p in perms:
      v = v + lax.gather(v, p[:, None], gdims, (1,),
                         mode=lax.GatherScatterMode.PROMISE_IN_BOUNDS)
    return v

  xs = (x0, x1, x2, x3)
  for q in range(2):
    cps = [pltpu.async_copy(xs[l].at[sidxa.at[q]], rbufs[l], gsem)
           for l in range(4)]
    for cp in cps:
      cp.wait()

    def usum(g, cy):
      for k in range(8):
        p = g * 8 + k
        for h in (0, 16):
          r0[p, pl.ds(h, 16)] = (r0[p, pl.ds(h, 16)] + r1[p, pl.ds(h, 16)]
                                 + r2[p, pl.ds(h, 16)] + r3[p, pl.ds(h, 16)])
      return cy

    lax.fori_loop(0, 16, usum, 0)
    cps = [pltpu.async_copy(xs[l].at[didxa.at[q]], rbufs[l + 1], gsem)
           for l in range(3)]
    for cp in cps:
      cp.wait()

    def dot1(t, cy, q=q):
      ilo = r1[t, pl.ds(0, 16)] + r2[t, pl.ds(0, 16)] + r3[t, pl.ds(0, 16)]
      ihi = r1[t, pl.ds(16, 16)] + r2[t, pl.ds(16, 16)] + r3[t, pl.ds(16, 16)]
      hs = _hsum(r0[t, pl.ds(0, 16)] * ilo + r0[t, pl.ds(16, 16)] * ihi)
      base = q * 128 + (t & ~15)
      av = gbuf[pl.ds(base, 16)]
      gbuf[pl.ds(base, 16)] = jnp.where(lane == (t & 15), hs, av)
      return cy

    lax.fori_loop(0, 128, dot1, 0)
    pltpu.async_copy(xs[3].at[didxa.at[q]], r1, gsem).wait()

    def dot2(t, cy, q=q):
      hs = _hsum(r0[t, pl.ds(0, 16)] * r1[t, pl.ds(0, 16)]
                 + r0[t, pl.ds(16, 16)] * r1[t, pl.ds(16, 16)])
      base = q * 128 + (t & ~15)
      av = gbuf[pl.ds(base, 16)]
      gbuf[pl.ds(base, 16)] = av + jnp.where(lane == (t & 15), hs, 0.0)
      return cy

    lax.fori_loop(0, 128, dot2, 0)

  def gscale(g, cy):
    gbuf[pl.ds(g * 16, 16)] = gbuf[pl.ds(g * 16, 16)] * 0.0625
    return cy

  lax.fori_loop(0, 16, gscale, 0)
  pltpu.sync_copy(gbuf, gpart.at[c, 0, pl.ds(s * 256, 256)])


def _make_kernel():
  mesh = plsc.VectorSubcoreMesh(core_axis_name="c", subcore_axis_name="s")
  out_type = [
      jax.ShapeDtypeStruct((2 * _RH, _H), jnp.float32),
      jax.ShapeDtypeStruct((2 * _RH, _H), jnp.float32),
      jax.ShapeDtypeStruct((2 * _RH, _H), jnp.float32),
      jax.ShapeDtypeStruct((2, 1, _B), jnp.float32),
  ]
  scratch = [
      pltpu.VMEM_SHARED((_RH, _H), jnp.float32),
      pltpu.VMEM((_CH, _H), jnp.float32),
      pltpu.VMEM((_CH, _H), jnp.float32),
      pltpu.VMEM((_CH, _H), jnp.float32),
      pltpu.VMEM((_CH, _H), jnp.float32),
      pltpu.VMEM((_GCH, 128), jnp.int32),
      pltpu.VMEM((_GCH, 128), jnp.int32),
      pltpu.VMEM((_GCH * _CH,), jnp.float32),
      pltpu.VMEM((_GCH, 128), jnp.int32),
      pltpu.VMEM((_GCH, 128), jnp.int32),
      pltpu.VMEM((_GCH * _CH,), jnp.float32),
      pltpu.VMEM((256,), jnp.float32),
      pltpu.SemaphoreType.DMA,
      pltpu.SemaphoreType.DMA,
      pltpu.SemaphoreType.DMA,
  ]
  return pl.kernel(_sc_body, out_type=out_type, mesh=mesh,
                   scratch_types=scratch,
                   compiler_params=pltpu.CompilerParams(
                       use_tc_tiling_on_sc=False))


_KERNEL = _make_kernel()


@jax.jit
def kernel(user_emb, item_emb, edge_index, edge_weight, users, items):
  all_emb = jnp.concatenate([user_emb, item_emb], axis=0)
  zpad = jnp.zeros((_RH - _N_NODES, _H), jnp.float32)
  x0 = jnp.concatenate(
      [all_emb[:, :_H], zpad, all_emb[:, _H:], zpad], axis=0)
  pad = _EPAD - _E
  srcp = jnp.concatenate(
      [edge_index[0], jnp.zeros((pad,), jnp.int32)]).reshape(_EPAD // 128, 128)
  dstp = jnp.concatenate(
      [edge_index[1], jnp.zeros((pad,), jnp.int32)]).reshape(_EPAD // 128, 128)
  wp = jnp.concatenate([edge_weight, jnp.zeros((pad,), jnp.float32)])
  usr = users.reshape(_B // 128, 128)
  itm = items.reshape(_B // 128, 128)
  zer = jnp.zeros((_TR, _H), jnp.float32)
  _, _, _, gpart = _KERNEL(x0, srcp, dstp, wp, usr, itm, zer)
  return gpart[0, 0] + gpart[1, 0]
